# baseline (speedup 1.0000x reference)
import functools

import jax
import jax.numpy as jnp
from jax.experimental import pallas as pl
from jax.experimental.pallas import tpu as pltpu
from jax.experimental.pallas import tpu_sc as plsc

INNER = 64
MODEL = 512
_GW = 64
_PP_V = 32768
_MT = 3200
_K = 8


def _pp_body(t_t_ref, o_ref):
    blk = t_t_ref[...]
    eye = jnp.eye(INNER, dtype=jnp.float32)
    t = jax.lax.dot_general(blk, eye, (((0,), (0,)), ((), ())),
                            preferred_element_type=jnp.float32)
    o_ref[:, :INNER] = t


def _tc_pack_table(t_t):
    v = t_t.shape[1]
    grid = (v + _PP_V - 1) // _PP_V
    return pl.pallas_call(
        _pp_body,
        grid=(grid,),
        in_specs=[pl.BlockSpec((INNER, _PP_V), lambda i: (0, i))],
        out_specs=pl.BlockSpec((_PP_V, 128), lambda i: (i, 0)),
        out_shape=jax.ShapeDtypeStruct((v, 128), jnp.float32),
    )(t_t)


def _sc_gather(table, idx, chunk, nc):
    mesh = plsc.VectorSubcoreMesh(core_axis_name="c", subcore_axis_name="s")
    base = chunk * nc // _GW
    nhb = nc // 2 // _GW

    @functools.partial(
        pl.kernel,
        out_type=jax.ShapeDtypeStruct((nc, INNER), table.dtype),
        mesh=mesh,
        scratch_types=[pltpu.VMEM((2 * _GW,), jnp.int32)],
        compiler_params=pltpu.CompilerParams(
            use_tc_tiling_on_sc=False, needs_layout_passes=False),
    )
    def gather_kernel(table_hbm, idx_hbm, out_hbm, ilv_ref):
        def body(lo_ref, hi_ref, o_vmem):
            @pl.loop(0, _GW // 16)
            def _(k):
                lo = lo_ref[0, pl.ds(k * 16, 16)]
                hi = hi_ref[0, pl.ds(k * 16, 16)]
                pos = jax.lax.iota(jnp.int32, 16) * 2 + k * 32
                plsc.store_scatter(ilv_ref, [pos], lo)
                plsc.store_scatter(ilv_ref, [pos + 1], hi)
            pltpu.sync_copy(table_hbm.at[ilv_ref], o_vmem)

        pltpu.emit_pipeline(
            body,
            grid=(nc // (2 * _GW),),
            in_specs=[
                pl.BlockSpec((1, _GW), lambda i: (0, base + i)),
                pl.BlockSpec((1, _GW), lambda i: (0, base + nhb + i)),
            ],
            out_specs=[pl.BlockSpec((2 * _GW, INNER), lambda i: (i, 0))],
            core_axis_name=("c", "s"),
            dimension_semantics=(pltpu.PARALLEL,),
        )(idx_hbm, idx_hbm, out_hbm)

    return gather_kernel(table, idx)


def _mm_core(h2_ref, p_ref, o_ref):
    half = pl.program_id(1)
    h2 = h2_ref[...]
    hs = jnp.where(half == 0, h2[:, :INNER], h2[:, INNER:])
    p = p_ref[...]
    o_ref[...] = jax.lax.dot_general(
        hs.astype(jnp.bfloat16), p.astype(jnp.bfloat16),
        (((1,), (1,)), ((), ())),
        preferred_element_type=jnp.float32)


def _mm_body_alias(o_any, h2_ref, p_ref, o_ref):
    del o_any
    _mm_core(h2_ref, p_ref, o_ref)


def _tc_project_chunk(out_prev, h2, proj_weight, chunk, n):
    n2 = h2.shape[0]
    nblk = n2 // _MT
    cbase = chunk * 2 * nblk
    hp_specs = [
        pl.BlockSpec((_MT, 128), lambda i, j: (i, 0)),
        pl.BlockSpec((MODEL, INNER), lambda i, j: (0, 0)),
    ]
    out_spec = pl.BlockSpec(
        (_MT, MODEL), lambda i, j: (cbase + j * nblk + i, 0))
    out_shape = jax.ShapeDtypeStruct((n, MODEL), jnp.float32)
    if out_prev is None:
        return pl.pallas_call(
            _mm_core,
            grid=(nblk, 2),
            in_specs=hp_specs,
            out_specs=out_spec,
            out_shape=out_shape,
        )(h2, proj_weight)
    return pl.pallas_call(
        _mm_body_alias,
        grid=(nblk, 2),
        in_specs=[pl.BlockSpec(memory_space=pltpu.MemorySpace.HBM)] + hp_specs,
        out_specs=out_spec,
        out_shape=out_shape,
        input_output_aliases={0: 0},
    )(out_prev, h2, proj_weight)


def kernel(x, embed_weight, proj_weight):
    b, l = x.shape
    v = embed_weight.shape[0]
    n = b * l
    nc = n // _K
    xf = (x.reshape(1, n)) * 2
    t_t = jnp.swapaxes(embed_weight, 0, 1)
    t2 = _tc_pack_table(t_t)
    tlin = t2.reshape(2 * v, INNER)
    hs = [_sc_gather(tlin, xf, c, nc) for c in range(_K)]
    out = None
    for c in range(_K):
        h2 = hs[c].reshape(nc // 2, 128)
        out = _tc_project_chunk(out, h2, proj_weight, c, n)
    return out.reshape(b, l, MODEL)


# speedup vs baseline: 1.0406x; 1.0406x over previous
import functools

import jax
import jax.numpy as jnp
from jax.experimental import pallas as pl
from jax.experimental.pallas import tpu as pltpu
from jax.experimental.pallas import tpu_sc as plsc

INNER = 64
MODEL = 512
_GW = 128
_PP_V = 32768
_MT = 4096
_K = 4


def _pp_body(t_t_ref, o_ref):
    blk = t_t_ref[...]
    eye = jnp.eye(INNER, dtype=jnp.float32)
    t = jax.lax.dot_general(blk, eye, (((0,), (0,)), ((), ())),
                            preferred_element_type=jnp.float32)
    o_ref[:, :INNER] = t


def _tc_pack_table(t_t):
    v = t_t.shape[1]
    grid = (v + _PP_V - 1) // _PP_V
    return pl.pallas_call(
        _pp_body,
        grid=(grid,),
        in_specs=[pl.BlockSpec((INNER, _PP_V), lambda i: (0, i))],
        out_specs=pl.BlockSpec((_PP_V, 128), lambda i: (i, 0)),
        out_shape=jax.ShapeDtypeStruct((v, 128), jnp.float32),
    )(t_t)


def _sc_gather(table, idx, chunk, nc):
    mesh = plsc.VectorSubcoreMesh(core_axis_name="c", subcore_axis_name="s")
    base = chunk * nc // _GW
    nhb = nc // 2 // _GW

    @functools.partial(
        pl.kernel,
        out_type=jax.ShapeDtypeStruct((nc, INNER), table.dtype),
        mesh=mesh,
        scratch_types=[pltpu.VMEM((2 * _GW,), jnp.int32)],
        compiler_params=pltpu.CompilerParams(
            use_tc_tiling_on_sc=False, needs_layout_passes=False),
    )
    def gather_kernel(table_hbm, idx_hbm, out_hbm, ilv_ref):
        def body(lo_ref, hi_ref, o_vmem):
            @pl.loop(0, _GW // 16)
            def _(k):
                lo = lo_ref[0, pl.ds(k * 16, 16)]
                hi = hi_ref[0, pl.ds(k * 16, 16)]
                pos = jax.lax.iota(jnp.int32, 16) * 2 + k * 32
                plsc.store_scatter(ilv_ref, [pos], lo)
                plsc.store_scatter(ilv_ref, [pos + 1], hi)
            pltpu.sync_copy(table_hbm.at[ilv_ref], o_vmem)

        pltpu.emit_pipeline(
            body,
            grid=(nc // (2 * _GW),),
            in_specs=[
                pl.BlockSpec((1, _GW), lambda i: (0, base + i)),
                pl.BlockSpec((1, _GW), lambda i: (0, base + nhb + i)),
            ],
            out_specs=[pl.BlockSpec((2 * _GW, INNER), lambda i: (i, 0))],
            core_axis_name=("c", "s"),
            dimension_semantics=(pltpu.PARALLEL,),
        )(idx_hbm, idx_hbm, out_hbm)

    return gather_kernel(table, idx)


def _mm_core(h2_ref, p_ref, o_ref):
    half = pl.program_id(1)
    h2 = h2_ref[...]
    hs = jnp.where(half == 0, h2[:, :INNER], h2[:, INNER:])
    p = p_ref[...]
    o_ref[...] = jax.lax.dot_general(
        hs.astype(jnp.bfloat16), p.astype(jnp.bfloat16),
        (((1,), (1,)), ((), ())),
        preferred_element_type=jnp.float32)


def _mm_body_alias(o_any, h2_ref, p_ref, o_ref):
    del o_any
    _mm_core(h2_ref, p_ref, o_ref)


def _tc_project_chunk(out_prev, h2, proj_weight, chunk, n):
    n2 = h2.shape[0]
    nblk = n2 // _MT
    cbase = chunk * 2 * nblk
    hp_specs = [
        pl.BlockSpec((_MT, 128), lambda i, j: (i, 0)),
        pl.BlockSpec((MODEL, INNER), lambda i, j: (0, 0)),
    ]
    out_spec = pl.BlockSpec(
        (_MT, MODEL), lambda i, j: (cbase + j * nblk + i, 0))
    out_shape = jax.ShapeDtypeStruct((n, MODEL), jnp.float32)
    if out_prev is None:
        return pl.pallas_call(
            _mm_core,
            grid=(nblk, 2),
            in_specs=hp_specs,
            out_specs=out_spec,
            out_shape=out_shape,
        )(h2, proj_weight)
    return pl.pallas_call(
        _mm_body_alias,
        grid=(nblk, 2),
        in_specs=[pl.BlockSpec(memory_space=pltpu.MemorySpace.HBM)] + hp_specs,
        out_specs=out_spec,
        out_shape=out_shape,
        input_output_aliases={0: 0},
    )(out_prev, h2, proj_weight)


def kernel(x, embed_weight, proj_weight):
    b, l = x.shape
    v = embed_weight.shape[0]
    n = b * l
    nc = n // _K
    xf = (x.reshape(1, n)) * 2
    t_t = jnp.swapaxes(embed_weight, 0, 1)
    t2 = _tc_pack_table(t_t)
    tlin = t2.reshape(2 * v, INNER)
    hs = [_sc_gather(tlin, xf, c, nc) for c in range(_K)]
    out = None
    for c in range(_K):
        h2 = hs[c].reshape(nc // 2, 128)
        out = _tc_project_chunk(out, h2, proj_weight, c, n)
    return out.reshape(b, l, MODEL)


# speedup vs baseline: 1.0449x; 1.0041x over previous
import functools

import jax
import jax.numpy as jnp
from jax.experimental import pallas as pl
from jax.experimental.pallas import tpu as pltpu
from jax.experimental.pallas import tpu_sc as plsc

INNER = 64
MODEL = 512
_GW = 128
_PP_V = 32768
_MT = 4096
_K = 4


def _pp_body(t_t_ref, o_ref):
    blk = t_t_ref[...]
    eye = jnp.eye(INNER, dtype=jnp.float32)
    t = jax.lax.dot_general(blk, eye, (((0,), (0,)), ((), ())),
                            preferred_element_type=jnp.float32)
    o_ref[:, :INNER] = t


def _tc_pack_table(t_t):
    v = t_t.shape[1]
    grid = (v + _PP_V - 1) // _PP_V
    return pl.pallas_call(
        _pp_body,
        grid=(grid,),
        in_specs=[pl.BlockSpec((INNER, _PP_V), lambda i: (0, i))],
        out_specs=pl.BlockSpec((_PP_V, 128), lambda i: (i, 0)),
        out_shape=jax.ShapeDtypeStruct((v, 128), jnp.float32),
    )(t_t)


def _sc_gather(table, idx, start, nc):
    mesh = plsc.VectorSubcoreMesh(core_axis_name="c", subcore_axis_name="s")
    base = start // _GW
    nhb = nc // 2 // _GW

    @functools.partial(
        pl.kernel,
        out_type=jax.ShapeDtypeStruct((nc, INNER), table.dtype),
        mesh=mesh,
        scratch_types=[pltpu.VMEM((2 * _GW,), jnp.int32)],
        compiler_params=pltpu.CompilerParams(
            use_tc_tiling_on_sc=False, needs_layout_passes=False),
    )
    def gather_kernel(table_hbm, idx_hbm, out_hbm, ilv_ref):
        def body(lo_ref, hi_ref, o_vmem):
            @pl.loop(0, _GW // 16)
            def _(k):
                lo = lo_ref[0, pl.ds(k * 16, 16)]
                hi = hi_ref[0, pl.ds(k * 16, 16)]
                pos = jax.lax.iota(jnp.int32, 16) * 2 + k * 32
                plsc.store_scatter(ilv_ref, [pos], lo)
                plsc.store_scatter(ilv_ref, [pos + 1], hi)
            pltpu.sync_copy(table_hbm.at[ilv_ref], o_vmem)

        pltpu.emit_pipeline(
            body,
            grid=(nc // (2 * _GW),),
            in_specs=[
                pl.BlockSpec((1, _GW), lambda i: (0, base + i)),
                pl.BlockSpec((1, _GW), lambda i: (0, base + nhb + i)),
            ],
            out_specs=[pl.BlockSpec((2 * _GW, INNER), lambda i: (i, 0))],
            core_axis_name=("c", "s"),
            dimension_semantics=(pltpu.PARALLEL,),
        )(idx_hbm, idx_hbm, out_hbm)

    return gather_kernel(table, idx)


def _mm_core(h2_ref, p_ref, o_ref):
    half = pl.program_id(1)
    h2 = h2_ref[...]
    hs = jnp.where(half == 0, h2[:, :INNER], h2[:, INNER:])
    p = p_ref[...]
    o_ref[...] = jax.lax.dot_general(
        hs.astype(jnp.bfloat16), p.astype(jnp.bfloat16),
        (((1,), (1,)), ((), ())),
        preferred_element_type=jnp.float32)


def _mm_body_alias(o_any, h2_ref, p_ref, o_ref):
    del o_any
    _mm_core(h2_ref, p_ref, o_ref)


def _tc_project_chunk(out_prev, h2, proj_weight, start, n, mt):
    n2 = h2.shape[0]
    nblk = n2 // mt
    cbase = start // mt
    hp_specs = [
        pl.BlockSpec((mt, 128), lambda i, j: (i, 0)),
        pl.BlockSpec((MODEL, INNER), lambda i, j: (0, 0)),
    ]
    out_spec = pl.BlockSpec(
        (mt, MODEL), lambda i, j: (cbase + j * nblk + i, 0))
    out_shape = jax.ShapeDtypeStruct((n, MODEL), jnp.float32)
    if out_prev is None:
        return pl.pallas_call(
            _mm_core,
            grid=(nblk, 2),
            in_specs=hp_specs,
            out_specs=out_spec,
            out_shape=out_shape,
        )(h2, proj_weight)
    return pl.pallas_call(
        _mm_body_alias,
        grid=(nblk, 2),
        in_specs=[pl.BlockSpec(memory_space=pltpu.MemorySpace.HBM)] + hp_specs,
        out_specs=out_spec,
        out_shape=out_shape,
        input_output_aliases={0: 0},
    )(out_prev, h2, proj_weight)


def kernel(x, embed_weight, proj_weight):
    b, l = x.shape
    v = embed_weight.shape[0]
    n = b * l
    chunks = [(0, n // 8, 3200), (n // 8, n // 8, 3200),
              (n // 4, n // 4, _MT), (n // 2, n // 4, _MT),
              (3 * n // 4, n // 4, _MT)]
    xf = (x.reshape(1, n)) * 2
    t_t = jnp.swapaxes(embed_weight, 0, 1)
    t2 = _tc_pack_table(t_t)
    tlin = t2.reshape(2 * v, INNER)
    hs = [_sc_gather(tlin, xf, s0, nc) for s0, nc, _ in chunks]
    out = None
    for h, (s0, nc, mt) in zip(hs, chunks):
        h2 = h.reshape(nc // 2, 128)
        out = _tc_project_chunk(out, h2, proj_weight, s0, n, mt)
    return out.reshape(b, l, MODEL)


# speedup vs baseline: 1.0656x; 1.0198x over previous
import functools

import jax
import jax.numpy as jnp
from jax.experimental import pallas as pl
from jax.experimental.pallas import tpu as pltpu
from jax.experimental.pallas import tpu_sc as plsc

INNER = 64
MODEL = 512
_GW = 128
_PP_V = 32768
_MT = 4096
_K = 4


def _pp_body(t_t_ref, o_ref):
    blk = t_t_ref[...]
    eye = jnp.eye(INNER, dtype=jnp.float32)
    t = jax.lax.dot_general(blk, eye, (((0,), (0,)), ((), ())),
                            preferred_element_type=jnp.float32)
    o_ref[:, :INNER] = t


def _tc_pack_table(t_t):
    v = t_t.shape[1]
    grid = (v + _PP_V - 1) // _PP_V
    return pl.pallas_call(
        _pp_body,
        grid=(grid,),
        in_specs=[pl.BlockSpec((INNER, _PP_V), lambda i: (0, i))],
        out_specs=pl.BlockSpec((_PP_V, 128), lambda i: (i, 0)),
        out_shape=jax.ShapeDtypeStruct((v, 128), jnp.float32),
    )(t_t)


def _sc_gather(table, idx, start, nc):
    mesh = plsc.VectorSubcoreMesh(core_axis_name="c", subcore_axis_name="s")
    base = start // _GW
    nhb = nc // 2 // _GW

    @functools.partial(
        pl.kernel,
        out_type=jax.ShapeDtypeStruct((nc, INNER), table.dtype),
        mesh=mesh,
        scratch_types=[pltpu.VMEM((2 * _GW,), jnp.int32)],
        compiler_params=pltpu.CompilerParams(
            use_tc_tiling_on_sc=False, needs_layout_passes=False),
    )
    def gather_kernel(table_hbm, idx_hbm, out_hbm, ilv_ref):
        def body(lo_ref, hi_ref, o_vmem):
            @pl.loop(0, _GW // 16)
            def _(k):
                lo = lo_ref[0, pl.ds(k * 16, 16)]
                hi = hi_ref[0, pl.ds(k * 16, 16)]
                pos = jax.lax.iota(jnp.int32, 16) * 2 + k * 32
                plsc.store_scatter(ilv_ref, [pos], lo)
                plsc.store_scatter(ilv_ref, [pos + 1], hi)
            pltpu.sync_copy(table_hbm.at[ilv_ref], o_vmem)

        pltpu.emit_pipeline(
            body,
            grid=(nc // (2 * _GW),),
            in_specs=[
                pl.BlockSpec((1, _GW), lambda i: (0, base + i)),
                pl.BlockSpec((1, _GW), lambda i: (0, base + nhb + i)),
            ],
            out_specs=[pl.BlockSpec((2 * _GW, INNER), lambda i: (i, 0))],
            core_axis_name=("c", "s"),
            dimension_semantics=(pltpu.PARALLEL,),
        )(idx_hbm, idx_hbm, out_hbm)

    return gather_kernel(table, idx)


def _mm_core(h2_ref, p_ref, o_ref):
    half = pl.program_id(1)
    h2 = h2_ref[...]
    hs = jnp.where(half == 0, h2[:, :INNER], h2[:, INNER:])
    p = p_ref[...]
    o_ref[...] = jax.lax.dot_general(
        hs.astype(jnp.bfloat16), p.astype(jnp.bfloat16),
        (((1,), (1,)), ((), ())),
        preferred_element_type=jnp.float32)


def _mm_body_alias(o_any, h2_ref, p_ref, o_ref):
    del o_any
    _mm_core(h2_ref, p_ref, o_ref)


def _tc_project_chunk(out_prev, h2, proj_weight, start, n, mt):
    n2 = h2.shape[0]
    nblk = n2 // mt
    cbase = start // mt
    hp_specs = [
        pl.BlockSpec((mt, 128), lambda i, j: (i, 0)),
        pl.BlockSpec((MODEL, INNER), lambda i, j: (0, 0)),
    ]
    out_spec = pl.BlockSpec(
        (mt, MODEL), lambda i, j: (cbase + j * nblk + i, 0))
    out_shape = jax.ShapeDtypeStruct((n, MODEL), jnp.float32)
    if out_prev is None:
        return pl.pallas_call(
            _mm_core,
            grid=(nblk, 2),
            in_specs=hp_specs,
            out_specs=out_spec,
            out_shape=out_shape,
        )(h2, proj_weight)
    return pl.pallas_call(
        _mm_body_alias,
        grid=(nblk, 2),
        in_specs=[pl.BlockSpec(memory_space=pltpu.MemorySpace.HBM)] + hp_specs,
        out_specs=out_spec,
        out_shape=out_shape,
        input_output_aliases={0: 0},
    )(out_prev, h2, proj_weight)


def kernel(x, embed_weight, proj_weight):
    b, l = x.shape
    v = embed_weight.shape[0]
    n = b * l
    chunks = [(0, n // 8, 6400), (n // 8, n // 8, 6400),
              (n // 4, n // 4, 6400), (n // 2, n // 4, 6400),
              (3 * n // 4, n // 4, 6400)]
    xf = (x.reshape(1, n)) * 2
    t_t = jnp.swapaxes(embed_weight, 0, 1)
    t2 = _tc_pack_table(t_t)
    tlin = t2.reshape(2 * v, INNER)
    hs = [_sc_gather(tlin, xf, s0, nc) for s0, nc, _ in chunks]
    out = None
    for h, (s0, nc, mt) in zip(hs, chunks):
        h2 = h.reshape(nc // 2, 128)
        out = _tc_project_chunk(out, h2, proj_weight, s0, n, mt)
    return out.reshape(b, l, MODEL)
